# baseline (speedup 1.0000x reference)
import jax
import jax.numpy as jnp
from jax.experimental import pallas as pl
from jax.experimental.pallas import tpu as pltpu

NUM_NODES = 2_000_000
NUM_MOVABLE = 1_800_000
NUM_ROWS = 2048

LANES = 128
R = 744
BLK = R * LANES
POS_ROWS = (2 * NUM_NODES) // LANES
GRID = -(-POS_ROWS // R)
MASK_LEFT_PAD = 128
MASK_BLK0 = (NUM_NODES - MASK_LEFT_PAD) // BLK
MASK_FRAME_ROWS = 14_136
MASK_LAST_BLK = 18
MOV_BLK_LO = MASK_BLK0
MOV_BLK_HI = (NUM_NODES + NUM_MOVABLE - 1) // BLK


def _body(pos_ref, mask_ref, out_ref):
    i = pl.program_id(0)

    @pl.when(jnp.logical_and(i >= MOV_BLK_LO, i <= MOV_BLK_HI))
    def _quantize():
        x = pos_ref[...]
        q = jnp.clip(jnp.round(x), 0.0, float(NUM_ROWS - 1))
        out_ref[...] = jnp.where(mask_ref[...], q, x)

    @pl.when(jnp.logical_or(i < MOV_BLK_LO, i > MOV_BLK_HI))
    def _copy():
        out_ref[...] = pos_ref[...]


def kernel(pos, mask):
    pos2 = pos.reshape(POS_ROWS, LANES)
    maskp = jnp.concatenate([
        jnp.zeros((MASK_LEFT_PAD,), jnp.bool_),
        mask,
        jnp.zeros((MASK_FRAME_ROWS * LANES - MASK_LEFT_PAD - NUM_MOVABLE,),
                  jnp.bool_),
    ]).reshape(MASK_FRAME_ROWS, LANES)

    out = pl.pallas_call(
        _body,
        grid=(GRID,),
        in_specs=[
            pl.BlockSpec((R, LANES), lambda i: (i, 0)),
            pl.BlockSpec((R, LANES),
                         lambda i: (jnp.clip(i - MASK_BLK0, 0, MASK_LAST_BLK), 0)),
        ],
        out_specs=pl.BlockSpec((R, LANES), lambda i: (i, 0)),
        out_shape=jax.ShapeDtypeStruct((POS_ROWS, LANES), jnp.float32),
    )(pos2, maskp)
    return out.reshape(2 * NUM_NODES)
